# Initial kernel scaffold; baseline (speedup 1.0000x reference)
import jax
import jax.numpy as jnp
from jax.experimental import pallas as pl


def kernel(features_s, features_t, edge_index, theta1, theta2, W_gnn, b_gnn, W1, b1, W2, b2):
    raise NotImplementedError("write your pallas kernel here")


# speedup vs baseline: 2.9089x; 2.9089x over previous
import functools

import jax
import jax.numpy as jnp
from jax import lax
from jax.experimental import pallas as pl
from jax.experimental.pallas import tpu as pltpu
from jax.experimental.pallas import tpu_sc as plsc

N_NODES = 10000
D_FEAT = 512
CHUNK = 128
N_CHUNKS = D_FEAT // CHUNK
N_EDGES = 160000

NUM_SC = 2
NUM_TILES = 16
BE = 128
TPB = -(-N_EDGES // (NUM_TILES * BE))
E_PAD = NUM_TILES * TPB * BE
ACC_PT = 632
N_ACC = ACC_PT * NUM_TILES
OUT_PT = 624
OUT_TAIL = N_NODES - OUT_PT * NUM_TILES


def _sc_segment_sum(fc0, fc1, fc2, fc3, srcp, dstp, zeros_hbm):
    mesh = plsc.VectorSubcoreMesh(core_axis_name="c", subcore_axis_name="s")

    @functools.partial(
        pl.kernel,
        mesh=mesh,
        out_type=jax.ShapeDtypeStruct((N_CHUNKS, N_NODES, CHUNK), jnp.float32),
        scratch_types=[
            pltpu.VMEM((BE,), jnp.int32),
            pltpu.VMEM((BE,), jnp.int32),
            pltpu.VMEM((BE, CHUNK), jnp.float32),
            pltpu.VMEM_SHARED((N_ACC, CHUNK), jnp.float32),
            pltpu.SemaphoreType.DMA,
        ],
    )
    def seg_sum(fc0_h, fc1_h, fc2_h, fc3_h, src_h, dst_h, zeros_h,
                out_h, sidx, didx, rows, acc, sem):
        c = lax.axis_index("c")
        s = lax.axis_index("s")

        def process(fc_h, chunk_id):
            pltpu.sync_copy(zeros_h, acc.at[pl.ds(s * ACC_PT, ACC_PT)])
            plsc.subcore_barrier()

            def body(i, carry):
                base = pl.multiple_of((s * TPB + i) * BE, BE)
                pltpu.sync_copy(src_h.at[pl.ds(base, BE)], sidx)
                pltpu.async_copy(fc_h.at[sidx], rows, sem).wait()
                pltpu.sync_copy(dst_h.at[pl.ds(base, BE)], didx)
                pltpu.sync_copy(rows, acc.at[didx], add=True)
                return carry

            lax.fori_loop(0, TPB, body, 0)
            plsc.subcore_barrier()
            pltpu.sync_copy(
                acc.at[pl.ds(s * OUT_PT, OUT_PT)],
                out_h.at[chunk_id, pl.ds(s * OUT_PT, OUT_PT)],
            )

            @pl.when(s == NUM_TILES - 1)
            def _():
                pltpu.sync_copy(
                    acc.at[pl.ds(OUT_PT * NUM_TILES, OUT_TAIL)],
                    out_h.at[chunk_id, pl.ds(OUT_PT * NUM_TILES, OUT_TAIL)],
                )

            plsc.subcore_barrier()

        @pl.when(c == 0)
        def _():
            process(fc0_h, 0)
            process(fc1_h, 1)

        @pl.when(c == 1)
        def _():
            process(fc2_h, 2)
            process(fc3_h, 3)

    return seg_sum(fc0, fc1, fc2, fc3, srcp, dstp, zeros_hbm)


def _tc_dense_body(th_ref, feats_ref, hn_ref, wg_ref, wgr_ref, bg_ref,
                   w1_ref, b1_ref, w2_ref, b2_ref, out_ref):
    t1 = th_ref[0, 0]
    t2 = th_ref[0, 1]
    x = feats_ref[...]
    z1 = jnp.dot(x, wg_ref[...], preferred_element_type=jnp.float32) + bg_ref[...]
    z2 = bg_ref[...]
    for ci in range(N_CHUNKS):
        z2 = z2 + jnp.dot(hn_ref[ci], wgr_ref[ci],
                          preferred_element_type=jnp.float32)
    h = t1 * jnp.maximum(z1, 0.0) + t2 * jnp.maximum(z2, 0.0)
    m = jnp.maximum(
        jnp.dot(h, w1_ref[...], preferred_element_type=jnp.float32) + b1_ref[...],
        0.0)
    out_ref[...] = (jnp.dot(m, w2_ref[...], preferred_element_type=jnp.float32)
                    + b2_ref[...])


def _tc_dense(th, feats, hn, W_gnn, Wg_r, b_gnn, W1, b1, W2, b2):
    R = 1000
    grid = (N_NODES // R,)
    hid = W1.shape[1]
    ncls = W2.shape[1]
    return pl.pallas_call(
        _tc_dense_body,
        grid=grid,
        in_specs=[
            pl.BlockSpec((1, 128), lambda i: (0, 0)),
            pl.BlockSpec((R, D_FEAT), lambda i: (i, 0)),
            pl.BlockSpec((N_CHUNKS, R, CHUNK), lambda i: (0, i, 0)),
            pl.BlockSpec((D_FEAT, D_FEAT), lambda i: (0, 0)),
            pl.BlockSpec((N_CHUNKS, CHUNK, D_FEAT), lambda i: (0, 0, 0)),
            pl.BlockSpec((1, D_FEAT), lambda i: (0, 0)),
            pl.BlockSpec((D_FEAT, hid), lambda i: (0, 0)),
            pl.BlockSpec((1, hid), lambda i: (0, 0)),
            pl.BlockSpec((hid, ncls), lambda i: (0, 0)),
            pl.BlockSpec((1, ncls), lambda i: (0, 0)),
        ],
        out_specs=pl.BlockSpec((R, ncls), lambda i: (i, 0)),
        out_shape=jax.ShapeDtypeStruct((N_NODES, ncls), jnp.float32),
    )(th, feats, hn, W_gnn, Wg_r, b_gnn, W1, b1, W2, b2)


def kernel(features_s, features_t, edge_index, theta1, theta2,
           W_gnn, b_gnn, W1, b1, W2, b2):
    feats = jnp.concatenate([features_s, features_t], axis=0)
    fcs = [feats[:, i * CHUNK:(i + 1) * CHUNK] for i in range(N_CHUNKS)]

    pad = E_PAD - N_EDGES
    srcp = jnp.concatenate([edge_index[0], jnp.zeros((pad,), jnp.int32)])
    dstp = jnp.concatenate([edge_index[1],
                            jnp.full((pad,), N_NODES, jnp.int32)])
    zeros_hbm = jnp.zeros((ACC_PT, CHUNK), jnp.float32)

    hn = _sc_segment_sum(fcs[0], fcs[1], fcs[2], fcs[3], srcp, dstp, zeros_hbm)

    th = jnp.zeros((1, 128), jnp.float32)
    th = th.at[0, 0].set(theta1[0]).at[0, 1].set(theta2[0])
    Wg_r = W_gnn.reshape(N_CHUNKS, CHUNK, D_FEAT)
    y = _tc_dense(th, feats, hn, W_gnn, Wg_r, b_gnn.reshape(1, -1),
                  W1, b1.reshape(1, -1), W2, b2.reshape(1, -1))
    return (y[:features_s.shape[0]], y[features_s.shape[0]:])


# speedup vs baseline: 3.1751x; 1.0915x over previous
import functools

import jax
import jax.numpy as jnp
from jax import lax
from jax.experimental import pallas as pl
from jax.experimental.pallas import tpu as pltpu
from jax.experimental.pallas import tpu_sc as plsc

N_NODES = 10000
D_FEAT = 512
CHUNK = 128
N_CHUNKS = D_FEAT // CHUNK
CH_PER_SC = N_CHUNKS // 2
N_EDGES = 160000

NUM_TILES = 16
BE = 128
TPB = 80
E_PAD = NUM_TILES * TPB * BE
NB = 3
NI = 6
UNROLL = 6
STEPS = TPB // UNROLL * UNROLL
N_ACC = 10016
ZPT = 632
ZTAIL = N_ACC - 15 * ZPT
OUT_PT = 624
OUT_TAIL = N_NODES - OUT_PT * NUM_TILES


def _sc_segment_sum(fcs, srcp, dstp, zeros_hbm):
    mesh = plsc.VectorSubcoreMesh(core_axis_name="c", subcore_axis_name="s")

    @functools.partial(
        pl.kernel,
        mesh=mesh,
        out_type=jax.ShapeDtypeStruct((N_CHUNKS, N_NODES, CHUNK), jnp.float32),
        scratch_types=[
            [pltpu.VMEM((BE,), jnp.int32) for _ in range(NI)],
            [pltpu.VMEM((BE,), jnp.int32) for _ in range(NI)],
            [pltpu.VMEM((BE, CHUNK), jnp.float32) for _ in range(NB)],
            pltpu.VMEM_SHARED((N_ACC, CHUNK), jnp.float32),
            [pltpu.SemaphoreType.DMA for _ in range(NI)],
            [pltpu.SemaphoreType.DMA for _ in range(NB)],
            [pltpu.SemaphoreType.DMA for _ in range(NB)],
        ],
    )
    def seg_sum(*refs):
        fc_hs = refs[:N_CHUNKS]
        src_h, dst_h, zeros_h = refs[N_CHUNKS:N_CHUNKS + 3]
        out_h = refs[N_CHUNKS + 3]
        sidx = refs[N_CHUNKS + 4]
        didx = refs[N_CHUNKS + 5]
        rows = refs[N_CHUNKS + 6]
        acc = refs[N_CHUNKS + 7]
        isem = refs[N_CHUNKS + 8]
        gsem = refs[N_CHUNKS + 9]
        ssem = refs[N_CHUNKS + 10]

        c = lax.axis_index("c")
        s = lax.axis_index("s")

        def process(fc_h, chunk_id):
            @pl.when(s < NUM_TILES - 1)
            def _():
                pltpu.sync_copy(zeros_h, acc.at[pl.ds(s * ZPT, ZPT)])

            @pl.when(s == NUM_TILES - 1)
            def _():
                pltpu.sync_copy(zeros_h.at[pl.ds(0, ZTAIL)],
                                acc.at[pl.ds(15 * ZPT, ZTAIL)])

            plsc.subcore_barrier()

            def issue_idx(i, q):
                base = pl.multiple_of((s * TPB + i) * BE, BE)
                pltpu.async_copy(src_h.at[pl.ds(base, BE)], sidx[q], isem[q])
                pltpu.async_copy(dst_h.at[pl.ds(base, BE)], didx[q], isem[q])

            def wait_idx(q):
                pltpu.make_async_copy(src_h.at[pl.ds(0, BE)], sidx[q],
                                      isem[q]).wait()
                pltpu.make_async_copy(dst_h.at[pl.ds(0, BE)], didx[q],
                                      isem[q]).wait()

            def issue_gather(q, r):
                pltpu.async_copy(fc_h.at[sidx[q]], rows[r], gsem[r])

            def wait_gather(q, r):
                pltpu.make_async_copy(fc_h.at[sidx[q]], rows[r],
                                      gsem[r]).wait()

            def issue_scatter(q, r):
                pltpu.async_copy(rows[r], acc.at[didx[q]], ssem[r], add=True)

            def wait_scatter(q, r):
                pltpu.make_async_copy(rows[r], acc.at[didx[q]],
                                      ssem[r]).wait()

            def step(i, k, first=False, static_tail=False):
                wait_gather(k % NI, k % NB)
                issue_scatter(k % NI, k % NB)
                if not first:
                    wait_scatter((k - 1) % NI, (k - 1) % NB)
                if static_tail:
                    if i + 4 < TPB:
                        issue_idx(i + 4, (k + 4) % NI)
                    if i + 2 < TPB:
                        wait_idx((k + 2) % NI)
                        issue_gather((k + 2) % NI, (k + 2) % NB)
                else:
                    @pl.when(i + 4 < TPB)
                    def _():
                        issue_idx(i + 4, (k + 4) % NI)

                    wait_idx((k + 2) % NI)
                    issue_gather((k + 2) % NI, (k + 2) % NB)

            for j in range(4):
                issue_idx(j, j)
            for j in range(NB - 1):
                wait_idx(j)
                issue_gather(j, j)
            for i in range(UNROLL):
                step(i, i, first=(i == 0), static_tail=True)

            def outer(g, carry):
                for k in range(UNROLL):
                    step(g * UNROLL + k, k)
                return carry

            lax.fori_loop(1, STEPS // UNROLL, outer, 0)
            for i in range(STEPS, TPB):
                step(i, i, static_tail=True)
            wait_scatter((TPB - 1) % NI, (TPB - 1) % NB)

            plsc.subcore_barrier()
            pltpu.sync_copy(
                acc.at[pl.ds(s * OUT_PT, OUT_PT)],
                out_h.at[chunk_id, pl.ds(s * OUT_PT, OUT_PT)],
            )

            @pl.when(s == NUM_TILES - 1)
            def _():
                pltpu.sync_copy(
                    acc.at[pl.ds(OUT_PT * NUM_TILES, OUT_TAIL)],
                    out_h.at[chunk_id, pl.ds(OUT_PT * NUM_TILES, OUT_TAIL)],
                )

            plsc.subcore_barrier()

        @pl.when(c == 0)
        def _():
            for k in range(CH_PER_SC):
                process(fc_hs[k], k)

        @pl.when(c == 1)
        def _():
            for k in range(CH_PER_SC, N_CHUNKS):
                process(fc_hs[k], k)

    return seg_sum(*fcs, srcp, dstp, zeros_hbm)


def _tc_dense_body(th_ref, feats_ref, hn_ref, wg_ref, bg_ref,
                   w1_ref, b1_ref, w2_ref, b2_ref, out_ref):
    t1 = th_ref[0, 0]
    t2 = th_ref[0, 1]
    x = feats_ref[...]
    z1 = jnp.dot(x, wg_ref[...], preferred_element_type=jnp.float32) + bg_ref[...]
    hcat = jnp.concatenate([hn_ref[ci] for ci in range(N_CHUNKS)], axis=1)
    z2 = jnp.dot(hcat, wg_ref[...], preferred_element_type=jnp.float32) + bg_ref[...]
    h = t1 * jnp.maximum(z1, 0.0) + t2 * jnp.maximum(z2, 0.0)
    m = jnp.maximum(
        jnp.dot(h, w1_ref[...], preferred_element_type=jnp.float32) + b1_ref[...],
        0.0)
    out_ref[...] = (jnp.dot(m, w2_ref[...], preferred_element_type=jnp.float32)
                    + b2_ref[...])


def _tc_dense(th, feats, hn, W_gnn, b_gnn, W1, b1, W2, b2):
    R = 1000
    grid = (N_NODES // R,)
    hid = W1.shape[1]
    ncls = W2.shape[1]
    return pl.pallas_call(
        _tc_dense_body,
        grid=grid,
        in_specs=[
            pl.BlockSpec((1, 128), lambda i: (0, 0)),
            pl.BlockSpec((R, D_FEAT), lambda i: (i, 0)),
            pl.BlockSpec((N_CHUNKS, R, CHUNK), lambda i: (0, i, 0)),
            pl.BlockSpec((D_FEAT, D_FEAT), lambda i: (0, 0)),
            pl.BlockSpec((1, D_FEAT), lambda i: (0, 0)),
            pl.BlockSpec((D_FEAT, hid), lambda i: (0, 0)),
            pl.BlockSpec((1, hid), lambda i: (0, 0)),
            pl.BlockSpec((hid, ncls), lambda i: (0, 0)),
            pl.BlockSpec((1, ncls), lambda i: (0, 0)),
        ],
        out_specs=pl.BlockSpec((R, ncls), lambda i: (i, 0)),
        out_shape=jax.ShapeDtypeStruct((N_NODES, ncls), jnp.float32),
    )(th, feats, hn, W_gnn, b_gnn, W1, b1, W2, b2)


def kernel(features_s, features_t, edge_index, theta1, theta2,
           W_gnn, b_gnn, W1, b1, W2, b2):
    feats = jnp.concatenate([features_s, features_t], axis=0)
    fcs = [feats[:, i * CHUNK:(i + 1) * CHUNK] for i in range(N_CHUNKS)]

    pad = E_PAD - N_EDGES
    srcp = jnp.concatenate([edge_index[0], jnp.zeros((pad,), jnp.int32)])
    dstp = jnp.concatenate([edge_index[1],
                            jnp.full((pad,), N_NODES, jnp.int32)])
    zeros_hbm = jnp.zeros((ZPT, CHUNK), jnp.float32)

    hn = _sc_segment_sum(fcs, srcp, dstp, zeros_hbm)

    th = jnp.zeros((1, 128), jnp.float32)
    th = th.at[0, 0].set(theta1[0]).at[0, 1].set(theta2[0])
    y = _tc_dense(th, feats, hn, W_gnn, b_gnn.reshape(1, -1),
                  W1, b1.reshape(1, -1), W2, b2.reshape(1, -1))
    return (y[:features_s.shape[0]], y[features_s.shape[0]:])


# speedup vs baseline: 7.2239x; 2.2752x over previous
import functools

import jax
import jax.numpy as jnp
from jax import lax
from jax.experimental import pallas as pl
from jax.experimental.pallas import tpu as pltpu
from jax.experimental.pallas import tpu_sc as plsc

N_NODES = 10000
D_FEAT = 512
CHUNK = 128
N_CHUNKS = D_FEAT // CHUNK
CH_PER_SC = N_CHUNKS // 2
N_EDGES = 160000

NUM_TILES = 16
BE = 128
TPB = 80
E_PAD = NUM_TILES * TPB * BE
NB = 3
NI = 6
GA = 2
IA = 4
SD = 1
UNROLL = 6
STEPS = TPB // UNROLL * UNROLL
N_ACC = 10016
ZPT = 632
ZTAIL = N_ACC - 15 * ZPT
OUT_PT = 624
OUT_TAIL = N_NODES - OUT_PT * NUM_TILES


def _sc_segment_sum(fcs, srcp, dstp, zeros_hbm):
    mesh = plsc.VectorSubcoreMesh(core_axis_name="c", subcore_axis_name="s")

    @functools.partial(
        pl.kernel,
        mesh=mesh,
        out_type=jax.ShapeDtypeStruct((N_CHUNKS, N_NODES, CHUNK), jnp.float32),
        scratch_types=[
            [pltpu.VMEM((BE,), jnp.int32) for _ in range(NI)],
            [pltpu.VMEM((BE,), jnp.int32) for _ in range(NI)],
            [pltpu.VMEM((BE, CHUNK), jnp.float32) for _ in range(NB)],
            pltpu.VMEM_SHARED((N_ACC, CHUNK), jnp.float32),
            [pltpu.SemaphoreType.DMA for _ in range(NI)],
            [pltpu.SemaphoreType.DMA for _ in range(NB)],
            [pltpu.SemaphoreType.DMA for _ in range(NB)],
        ],
    )
    def seg_sum(*refs):
        fc_hs = refs[:N_CHUNKS]
        src_h, dst_h, zeros_h = refs[N_CHUNKS:N_CHUNKS + 3]
        out_h = refs[N_CHUNKS + 3]
        sidx = refs[N_CHUNKS + 4]
        didx = refs[N_CHUNKS + 5]
        rows = refs[N_CHUNKS + 6]
        acc = refs[N_CHUNKS + 7]
        isem = refs[N_CHUNKS + 8]
        gsem = refs[N_CHUNKS + 9]
        ssem = refs[N_CHUNKS + 10]

        c = lax.axis_index("c")
        s = lax.axis_index("s")

        def process(fc_h, chunk_id):
            @pl.when(s < NUM_TILES - 1)
            def _():
                pltpu.sync_copy(zeros_h, acc.at[pl.ds(s * ZPT, ZPT)])

            @pl.when(s == NUM_TILES - 1)
            def _():
                pltpu.sync_copy(zeros_h.at[pl.ds(0, ZTAIL)],
                                acc.at[pl.ds(15 * ZPT, ZTAIL)])

            plsc.subcore_barrier()

            def issue_idx(i, q):
                base = pl.multiple_of((s * TPB + i) * BE, BE)
                pltpu.async_copy(src_h.at[pl.ds(base, BE)], sidx[q], isem[q])
                pltpu.async_copy(dst_h.at[pl.ds(base, BE)], didx[q], isem[q])

            def wait_idx(q):
                pltpu.make_async_copy(src_h.at[pl.ds(0, BE)], sidx[q],
                                      isem[q]).wait()
                pltpu.make_async_copy(dst_h.at[pl.ds(0, BE)], didx[q],
                                      isem[q]).wait()

            def issue_gather(q, r):
                pltpu.async_copy(fc_h.at[sidx[q]], rows[r], gsem[r])

            def wait_gather(q, r):
                pltpu.make_async_copy(fc_h.at[sidx[q]], rows[r],
                                      gsem[r]).wait()

            def issue_scatter(q, r):
                pltpu.async_copy(rows[r], acc.at[didx[q]], ssem[r], add=True)

            def wait_scatter(q, r):
                pltpu.make_async_copy(rows[r], acc.at[didx[q]],
                                      ssem[r]).wait()

            def step(i, k, static=False):
                wait_gather(k % NI, k % NB)
                issue_scatter(k % NI, k % NB)
                if not static or i - SD >= 0:
                    wait_scatter((k - SD) % NI, (k - SD) % NB)
                if static:
                    if i + IA < TPB:
                        issue_idx(i + IA, (k + IA) % NI)
                    if i + GA < TPB:
                        wait_idx((k + GA) % NI)
                        issue_gather((k + GA) % NI, (k + GA) % NB)
                else:
                    @pl.when(i + IA < TPB)
                    def _():
                        issue_idx(i + IA, (k + IA) % NI)

                    wait_idx((k + GA) % NI)
                    issue_gather((k + GA) % NI, (k + GA) % NB)

            for j in range(IA):
                issue_idx(j, j)
            for j in range(GA):
                wait_idx(j)
                issue_gather(j, j)
            for i in range(UNROLL):
                step(i, i, static=True)

            def outer(g, carry):
                for k in range(UNROLL):
                    step(g * UNROLL + k, k)
                return carry

            lax.fori_loop(1, STEPS // UNROLL, outer, 0)
            for i in range(STEPS, TPB):
                step(i, i, static=True)
            for i in range(TPB - SD, TPB):
                wait_scatter(i % NI, i % NB)

            plsc.subcore_barrier()
            pltpu.sync_copy(
                acc.at[pl.ds(s * OUT_PT, OUT_PT)],
                out_h.at[chunk_id, pl.ds(s * OUT_PT, OUT_PT)],
            )

            @pl.when(s == NUM_TILES - 1)
            def _():
                pltpu.sync_copy(
                    acc.at[pl.ds(OUT_PT * NUM_TILES, OUT_TAIL)],
                    out_h.at[chunk_id, pl.ds(OUT_PT * NUM_TILES, OUT_TAIL)],
                )

            plsc.subcore_barrier()

        @pl.when(c == 0)
        def _():
            for k in range(CH_PER_SC):
                process(fc_hs[k], k)

        @pl.when(c == 1)
        def _():
            for k in range(CH_PER_SC, N_CHUNKS):
                process(fc_hs[k], k)

    return seg_sum(*fcs, srcp, dstp, zeros_hbm)


def _tc_dense_body(th_ref, feats_ref, hn_ref, wg_ref, bg_ref,
                   w1_ref, b1_ref, w2_ref, b2_ref, out_ref):
    t1 = th_ref[0, 0]
    t2 = th_ref[0, 1]
    x = feats_ref[...]
    z1 = jnp.dot(x, wg_ref[...], preferred_element_type=jnp.float32) + bg_ref[...]
    hcat = jnp.concatenate([hn_ref[ci] for ci in range(N_CHUNKS)], axis=1)
    z2 = jnp.dot(hcat, wg_ref[...], preferred_element_type=jnp.float32) + bg_ref[...]
    h = t1 * jnp.maximum(z1, 0.0) + t2 * jnp.maximum(z2, 0.0)
    m = jnp.maximum(
        jnp.dot(h, w1_ref[...], preferred_element_type=jnp.float32) + b1_ref[...],
        0.0)
    out_ref[...] = (jnp.dot(m, w2_ref[...], preferred_element_type=jnp.float32)
                    + b2_ref[...])


def _tc_dense(th, feats, hn, W_gnn, b_gnn, W1, b1, W2, b2):
    R = 1000
    grid = (N_NODES // R,)
    hid = W1.shape[1]
    ncls = W2.shape[1]
    return pl.pallas_call(
        _tc_dense_body,
        grid=grid,
        in_specs=[
            pl.BlockSpec((1, 128), lambda i: (0, 0)),
            pl.BlockSpec((R, D_FEAT), lambda i: (i, 0)),
            pl.BlockSpec((N_CHUNKS, R, CHUNK), lambda i: (0, i, 0)),
            pl.BlockSpec((D_FEAT, D_FEAT), lambda i: (0, 0)),
            pl.BlockSpec((1, D_FEAT), lambda i: (0, 0)),
            pl.BlockSpec((D_FEAT, hid), lambda i: (0, 0)),
            pl.BlockSpec((1, hid), lambda i: (0, 0)),
            pl.BlockSpec((hid, ncls), lambda i: (0, 0)),
            pl.BlockSpec((1, ncls), lambda i: (0, 0)),
        ],
        out_specs=pl.BlockSpec((R, ncls), lambda i: (i, 0)),
        out_shape=jax.ShapeDtypeStruct((N_NODES, ncls), jnp.float32),
    )(th, feats, hn, W_gnn, b_gnn, W1, b1, W2, b2)


def kernel(features_s, features_t, edge_index, theta1, theta2,
           W_gnn, b_gnn, W1, b1, W2, b2):
    feats = jnp.concatenate([features_s, features_t], axis=0)
    fcs = [feats[:, i * CHUNK:(i + 1) * CHUNK] for i in range(N_CHUNKS)]

    pad = E_PAD - N_EDGES
    pidx = jnp.arange(pad, dtype=jnp.int32)
    srcp = jnp.concatenate([edge_index[0], pidx % N_NODES])
    dstp = jnp.concatenate([edge_index[1],
                            N_NODES + (pidx % (N_ACC - N_NODES))])
    zeros_hbm = jnp.zeros((ZPT, CHUNK), jnp.float32)

    hn = _sc_segment_sum(fcs, srcp, dstp, zeros_hbm)

    th = jnp.zeros((1, 128), jnp.float32)
    th = th.at[0, 0].set(theta1[0]).at[0, 1].set(theta2[0])
    y = _tc_dense(th, feats, hn, W_gnn, b_gnn.reshape(1, -1),
                  W1, b1.reshape(1, -1), W2, b2.reshape(1, -1))
    return (y[:features_s.shape[0]], y[features_s.shape[0]:])


# speedup vs baseline: 7.3165x; 1.0128x over previous
import functools

import jax
import jax.numpy as jnp
from jax import lax
from jax.experimental import pallas as pl
from jax.experimental.pallas import tpu as pltpu
from jax.experimental.pallas import tpu_sc as plsc

N_NODES = 10000
D_FEAT = 512
CHUNK = 128
N_CHUNKS = D_FEAT // CHUNK
CH_PER_SC = N_CHUNKS // 2
N_EDGES = 160000

NUM_TILES = 16
BE = 128
TPB = 80
E_PAD = NUM_TILES * TPB * BE
NB = 3
NI = 6
GA = 2
IA = 4
SD = 1
UNROLL = 6
STEPS = TPB // UNROLL * UNROLL
N_ACC = 10016
ZPT = 632
ZTAIL = N_ACC - 15 * ZPT
OUT_PT = 624
OUT_TAIL = N_NODES - OUT_PT * NUM_TILES


def _sc_segment_sum(fcs, srcp, dstp, zeros_hbm):
    mesh = plsc.VectorSubcoreMesh(core_axis_name="c", subcore_axis_name="s")

    @functools.partial(
        pl.kernel,
        mesh=mesh,
        out_type=jax.ShapeDtypeStruct((N_CHUNKS, N_NODES, CHUNK), jnp.float32),
        scratch_types=[
            [pltpu.VMEM((BE,), jnp.int32) for _ in range(NI)],
            [pltpu.VMEM((BE,), jnp.int32) for _ in range(NI)],
            [pltpu.VMEM((BE, CHUNK), jnp.float32) for _ in range(NB)],
            pltpu.VMEM_SHARED((N_ACC, CHUNK), jnp.float32),
            [pltpu.SemaphoreType.DMA for _ in range(NI)],
            [pltpu.SemaphoreType.DMA for _ in range(NB)],
            [pltpu.SemaphoreType.DMA for _ in range(NB)],
        ],
    )
    def seg_sum(*refs):
        fc_hs = refs[:N_CHUNKS]
        src_h, dst_h, zeros_h = refs[N_CHUNKS:N_CHUNKS + 3]
        out_h = refs[N_CHUNKS + 3]
        sidx = refs[N_CHUNKS + 4]
        didx = refs[N_CHUNKS + 5]
        rows = refs[N_CHUNKS + 6]
        acc = refs[N_CHUNKS + 7]
        isem = refs[N_CHUNKS + 8]
        gsem = refs[N_CHUNKS + 9]
        ssem = refs[N_CHUNKS + 10]

        c = lax.axis_index("c")
        s = lax.axis_index("s")

        def process(fc_h, chunk_id):
            def issue_idx(i, q):
                base = pl.multiple_of((s * TPB + i) * BE, BE)
                pltpu.async_copy(src_h.at[pl.ds(base, BE)], sidx[q], isem[q])
                pltpu.async_copy(dst_h.at[pl.ds(base, BE)], didx[q], isem[q])

            def wait_idx(q):
                pltpu.make_async_copy(src_h.at[pl.ds(0, BE)], sidx[q],
                                      isem[q]).wait()
                pltpu.make_async_copy(dst_h.at[pl.ds(0, BE)], didx[q],
                                      isem[q]).wait()

            def issue_gather(q, r):
                pltpu.async_copy(fc_h.at[sidx[q]], rows[r], gsem[r])

            def wait_gather(q, r):
                pltpu.make_async_copy(fc_h.at[sidx[q]], rows[r],
                                      gsem[r]).wait()

            def issue_scatter(q, r):
                pltpu.async_copy(rows[r], acc.at[didx[q]], ssem[r], add=True)

            def wait_scatter(q, r):
                pltpu.make_async_copy(rows[r], acc.at[didx[q]],
                                      ssem[r]).wait()

            def step(i, k, static=False):
                wait_gather(k % NI, k % NB)
                issue_scatter(k % NI, k % NB)
                if not static or i - SD >= 0:
                    wait_scatter((k - SD) % NI, (k - SD) % NB)
                if static:
                    if i + IA < TPB:
                        issue_idx(i + IA, (k + IA) % NI)
                    if i + GA < TPB:
                        wait_idx((k + GA) % NI)
                        issue_gather((k + GA) % NI, (k + GA) % NB)
                else:
                    @pl.when(i + IA < TPB)
                    def _():
                        issue_idx(i + IA, (k + IA) % NI)

                    wait_idx((k + GA) % NI)
                    issue_gather((k + GA) % NI, (k + GA) % NB)

            for j in range(IA):
                issue_idx(j, j)
            for j in range(GA):
                wait_idx(j)
                issue_gather(j, j)

            @pl.when(s < NUM_TILES - 1)
            def _():
                pltpu.sync_copy(zeros_h, acc.at[pl.ds(s * ZPT, ZPT)])

            @pl.when(s == NUM_TILES - 1)
            def _():
                pltpu.sync_copy(zeros_h.at[pl.ds(0, ZTAIL)],
                                acc.at[pl.ds(15 * ZPT, ZTAIL)])

            plsc.subcore_barrier()

            for i in range(UNROLL):
                step(i, i, static=True)

            def outer(g, carry):
                for k in range(UNROLL):
                    step(g * UNROLL + k, k)
                return carry

            lax.fori_loop(1, STEPS // UNROLL, outer, 0)
            for i in range(STEPS, TPB):
                step(i, i, static=True)
            for i in range(TPB - SD, TPB):
                wait_scatter(i % NI, i % NB)

            plsc.subcore_barrier()
            pltpu.sync_copy(
                acc.at[pl.ds(s * OUT_PT, OUT_PT)],
                out_h.at[chunk_id, pl.ds(s * OUT_PT, OUT_PT)],
            )

            @pl.when(s == NUM_TILES - 1)
            def _():
                pltpu.sync_copy(
                    acc.at[pl.ds(OUT_PT * NUM_TILES, OUT_TAIL)],
                    out_h.at[chunk_id, pl.ds(OUT_PT * NUM_TILES, OUT_TAIL)],
                )

            plsc.subcore_barrier()

        @pl.when(c == 0)
        def _():
            for k in range(CH_PER_SC):
                process(fc_hs[k], k)

        @pl.when(c == 1)
        def _():
            for k in range(CH_PER_SC, N_CHUNKS):
                process(fc_hs[k], k)

    return seg_sum(*fcs, srcp, dstp, zeros_hbm)


def _tc_pre_body(th_ref, feats_ref, wg_ref, bg_ref, w1_ref, out_ref):
    t1 = th_ref[0, 0]
    z1 = (jnp.dot(feats_ref[...], wg_ref[...],
                  preferred_element_type=jnp.float32) + bg_ref[...])
    out_ref[...] = jnp.dot(t1 * jnp.maximum(z1, 0.0), w1_ref[...],
                           preferred_element_type=jnp.float32)


def _tc_pre(th, feats, W_gnn, b_gnn, W1):
    R = 1000
    hid = W1.shape[1]
    return pl.pallas_call(
        _tc_pre_body,
        grid=(N_NODES // R,),
        in_specs=[
            pl.BlockSpec((1, 128), lambda i: (0, 0)),
            pl.BlockSpec((R, D_FEAT), lambda i: (i, 0)),
            pl.BlockSpec((D_FEAT, D_FEAT), lambda i: (0, 0)),
            pl.BlockSpec((1, D_FEAT), lambda i: (0, 0)),
            pl.BlockSpec((D_FEAT, hid), lambda i: (0, 0)),
        ],
        out_specs=pl.BlockSpec((R, hid), lambda i: (i, 0)),
        out_shape=jax.ShapeDtypeStruct((N_NODES, hid), jnp.float32),
    )(th, feats, W_gnn, b_gnn, W1)


def _tc_post_body(th_ref, p_ref, hn_ref, wg_ref, bg_ref,
                  w1_ref, b1_ref, w2_ref, b2_ref, out_ref):
    t2 = th_ref[0, 1]
    hcat = jnp.concatenate([hn_ref[ci] for ci in range(N_CHUNKS)], axis=1)
    z2 = jnp.dot(hcat, wg_ref[...], preferred_element_type=jnp.float32) + bg_ref[...]
    m = jnp.maximum(
        p_ref[...]
        + jnp.dot(t2 * jnp.maximum(z2, 0.0), w1_ref[...],
                  preferred_element_type=jnp.float32)
        + b1_ref[...],
        0.0)
    out_ref[...] = (jnp.dot(m, w2_ref[...], preferred_element_type=jnp.float32)
                    + b2_ref[...])


def _tc_post(th, p, hn, W_gnn, b_gnn, W1, b1, W2, b2):
    R = 1000
    grid = (N_NODES // R,)
    hid = W1.shape[1]
    ncls = W2.shape[1]
    return pl.pallas_call(
        _tc_post_body,
        grid=grid,
        in_specs=[
            pl.BlockSpec((1, 128), lambda i: (0, 0)),
            pl.BlockSpec((R, hid), lambda i: (i, 0)),
            pl.BlockSpec((N_CHUNKS, R, CHUNK), lambda i: (0, i, 0)),
            pl.BlockSpec((D_FEAT, D_FEAT), lambda i: (0, 0)),
            pl.BlockSpec((1, D_FEAT), lambda i: (0, 0)),
            pl.BlockSpec((D_FEAT, hid), lambda i: (0, 0)),
            pl.BlockSpec((1, hid), lambda i: (0, 0)),
            pl.BlockSpec((hid, ncls), lambda i: (0, 0)),
            pl.BlockSpec((1, ncls), lambda i: (0, 0)),
        ],
        out_specs=pl.BlockSpec((R, ncls), lambda i: (i, 0)),
        out_shape=jax.ShapeDtypeStruct((N_NODES, ncls), jnp.float32),
    )(th, p, hn, W_gnn, b_gnn, W1, b1, W2, b2)


def kernel(features_s, features_t, edge_index, theta1, theta2,
           W_gnn, b_gnn, W1, b1, W2, b2):
    feats = jnp.concatenate([features_s, features_t], axis=0)
    fcs = [feats[:, i * CHUNK:(i + 1) * CHUNK] for i in range(N_CHUNKS)]

    pad = E_PAD - N_EDGES
    pidx = jnp.arange(pad, dtype=jnp.int32)
    srcp = jnp.concatenate([edge_index[0], pidx % N_NODES])
    dstp = jnp.concatenate([edge_index[1],
                            N_NODES + (pidx % (N_ACC - N_NODES))])
    zeros_hbm = jnp.zeros((ZPT, CHUNK), jnp.float32)

    th = jnp.zeros((1, 128), jnp.float32)
    th = th.at[0, 0].set(theta1[0]).at[0, 1].set(theta2[0])
    p = _tc_pre(th, feats, W_gnn, b_gnn.reshape(1, -1), W1)
    hn = _sc_segment_sum(fcs, srcp, dstp, zeros_hbm)
    y = _tc_post(th, p, hn, W_gnn, b_gnn.reshape(1, -1),
                 W1, b1.reshape(1, -1), W2, b2.reshape(1, -1))
    return (y[:features_s.shape[0]], y[features_s.shape[0]:])
